# baseline (speedup 1.0000x reference)
import functools

import jax
import jax.numpy as jnp
from jax import lax
from jax.experimental import pallas as pl
from jax.experimental.pallas import tpu as pltpu
from jax.experimental.pallas import tpu_sc as plsc

EMBED_DIM = 64
BATCH = 16384
NUM_CORES = 2
NUM_SUBCORES = 16
NUM_WORKERS = NUM_CORES * NUM_SUBCORES
B_PER_W = BATCH // NUM_WORKERS
CHUNK = 128
N_CHUNKS = B_PER_W // CHUNK
FIRE = 16


@functools.partial(
    pl.kernel,
    out_type=jax.ShapeDtypeStruct((BATCH * EMBED_DIM,), jnp.float32),
    mesh=plsc.VectorSubcoreMesh(core_axis_name="c", subcore_axis_name="s"),
    scratch_types=[
        pltpu.VMEM((N_CHUNKS, CHUNK), jnp.int32),
        pltpu.VMEM((B_PER_W * EMBED_DIM,), jnp.float32),
        pltpu.SemaphoreType.DMA,
        pltpu.SemaphoreType.DMA,
    ],
)
def _sc_gather(idx_hbm, table_hbm, out_hbm, idx_v, rows_v, sem_i, sem):
    wid = lax.axis_index("s") * NUM_CORES + lax.axis_index("c")
    base = wid * B_PER_W
    pltpu.async_copy(idx_hbm.at[wid], idx_v, sem_i).wait()

    for c in range(N_CHUNKS):
        def body(step, c=c):
            v = idx_v[c, pl.ds(step * FIRE, FIRE)]
            cps = []
            for b in range(FIRE):
                i = step * FIRE + b
                cps.append(
                    pltpu.async_copy(
                        table_hbm.at[pl.ds(v[b] * EMBED_DIM, EMBED_DIM)],
                        rows_v.at[pl.ds((c * CHUNK + i) * EMBED_DIM, EMBED_DIM)],
                        sem,
                    )
                )
            for cp in cps:
                cp.wait()
        pl.loop(0, CHUNK // FIRE)(body)

    pltpu.async_copy(
        rows_v, out_hbm.at[pl.ds(base * EMBED_DIM, B_PER_W * EMBED_DIM)], sem_i
    ).wait()


def kernel(action_idx, embedding_weight):
    idx = action_idx.astype(jnp.int32).reshape(NUM_WORKERS, N_CHUNKS, CHUNK)
    flat = _sc_gather(idx, embedding_weight.reshape(-1))
    return flat.reshape(BATCH, EMBED_DIM)


# speedup vs baseline: 2.1723x; 2.1723x over previous
import functools

import jax
import jax.numpy as jnp
from jax import lax
from jax.experimental import pallas as pl
from jax.experimental.pallas import tpu as pltpu
from jax.experimental.pallas import tpu_sc as plsc

N_ROWS = 1000000
EMBED_DIM = 64
BATCH = 16384
NUM_CORES = 2
NUM_SUBCORES = 16
NUM_WORKERS = NUM_CORES * NUM_SUBCORES
B_PER_W = BATCH // NUM_WORKERS
LANE = 16
CHUNK = 128
N_CHUNKS = B_PER_W // CHUNK
FIRE = 16
RING = 4


@functools.partial(
    pl.kernel,
    out_type=jax.ShapeDtypeStruct((BATCH, EMBED_DIM), jnp.float32),
    mesh=plsc.VectorSubcoreMesh(core_axis_name="c", subcore_axis_name="s"),
    compiler_params=pltpu.CompilerParams(needs_layout_passes=False),
    scratch_types=[
        pltpu.VMEM((N_CHUNKS, CHUNK), jnp.int32),
        pltpu.VMEM((EMBED_DIM, CHUNK), jnp.float32),
        pltpu.VMEM((EMBED_DIM, CHUNK), jnp.float32),
        pltpu.VMEM((EMBED_DIM, CHUNK), jnp.float32),
        pltpu.VMEM((EMBED_DIM, CHUNK), jnp.float32),
        pltpu.VMEM((B_PER_W, EMBED_DIM), jnp.float32),
        pltpu.SemaphoreType.DMA,
        pltpu.SemaphoreType.DMA,
    ],
)
def _sc_gather(idx_hbm, table_t_hbm, out_hbm, idx_v, col0, col1, col2, col3,
               rows_v, sem_i, sem):
    cols = (col0, col1, col2, col3)
    wid = lax.axis_index("s") * NUM_CORES + lax.axis_index("c")
    base = wid * B_PER_W

    pltpu.async_copy(idx_hbm.at[wid], idx_v, sem_i).wait()

    def body(step, c):
        p = step * FIRE
        jv = idx_v[c, pl.ds(p, FIRE)]
        for q in range(FIRE // RING):
            cps = []
            for b in range(RING):
                j = jv[q * RING + b]
                col = pl.multiple_of((j >> 7) * CHUNK, CHUNK)
                cps.append(
                    pltpu.async_copy(
                        table_t_hbm.at[:, pl.ds(col, CHUNK)], cols[b], sem
                    )
                )
            for b in range(RING):
                cps[b].wait()
                j = jv[q * RING + b]
                jl = jnp.full((LANE,), j & 127, jnp.int32)
                row = c * CHUNK + p + q * RING + b
                for k in range(EMBED_DIM // LANE):
                    ev = lax.iota(jnp.int32, LANE) + (k * LANE)
                    rows_v[row, pl.ds(k * LANE, LANE)] = plsc.load_gather(
                        cols[b], [ev, jl]
                    )
    for c in range(N_CHUNKS):
        pl.loop(0, CHUNK // FIRE)(functools.partial(body, c=c))

    pltpu.async_copy(rows_v, out_hbm.at[pl.ds(base, B_PER_W)], sem_i).wait()


def kernel(action_idx, embedding_weight):
    idx = action_idx.astype(jnp.int32).reshape(NUM_WORKERS, N_CHUNKS, CHUNK)
    return _sc_gather(idx, embedding_weight.T)
